# baseline (speedup 1.0000x reference)
import jax
import jax.numpy as jnp
import numpy as np
from jax.experimental import pallas as pl
from jax.experimental.pallas import tpu as pltpu

_TOKEN_BLOCK = 512


def _routing_consts(n: int, e: int):
    rk = jax.random.key(42)
    k1, k2, k3, k4 = jax.random.split(rk, 4)
    u = jax.random.uniform(k1, (n, e), dtype=jnp.float32)
    g1 = -jnp.log(-jnp.log(u + 1e-08) + 1e-08)
    u2 = jax.random.uniform(k2, (n,), jnp.float32)
    g3 = jax.random.gumbel(k3, (n, e), jnp.float32)
    ex = jax.random.randint(k4, (n,), 0, e)
    return (g1, u2.reshape(n, 1), g3, ex.reshape(n, 1).astype(jnp.int32))


_CONST_CACHE = {}
try:
    _CONST_CACHE[(8192, 64)] = tuple(
        np.asarray(a) for a in _routing_consts(8192, 64))
except Exception:
    pass


def _routing_body(x_ref, wr_ref, br_ref, w1_ref, b1_ref, w2_ref, b2_ref,
                  g1_ref, g3_ref, u2_ref, ex_ref,
                  w_out, idx_out, noisy_out, explor_out, ent_out):
    e = wr_ref.shape[0]
    xb = x_ref[...]
    dims = (((1,), (1,)), ((), ()))
    logits = jax.lax.dot_general(xb, wr_ref[...], dims,
                                 preferred_element_type=jnp.float32) + br_ref[...]
    h = jax.lax.dot_general(xb, w1_ref[...], dims,
                            preferred_element_type=jnp.float32) + b1_ref[...]
    h = jnp.maximum(h, 0.0)
    epre = jax.lax.dot_general(h, w2_ref[...], dims,
                               preferred_element_type=jnp.float32)[:, 0:1] + b2_ref[...]
    explor = jax.nn.sigmoid(epre)

    noisy = logits + g1_ref[...]
    m = jnp.max(noisy, axis=-1, keepdims=True)
    ee = jnp.exp(noisy - m)
    p = ee / jnp.sum(ee, axis=-1, keepdims=True)
    logp = jnp.log(p + 1e-08)
    ent = -jnp.sum(p * logp, axis=-1)

    score = logp + g3_ref[...]
    smax = jnp.max(score, axis=-1, keepdims=True)
    lane = jax.lax.broadcasted_iota(jnp.int32, score.shape, 1)
    exploit = jnp.min(jnp.where(score >= smax, lane, e), axis=-1, keepdims=True)
    mask = u2_ref[...] < 1.0 - explor * 0.1
    idx = jnp.where(mask, exploit, ex_ref[...])

    w_out[...] = p
    idx_out[...] = idx
    noisy_out[...] = noisy
    explor_out[...] = explor
    ent_out[...] = jnp.sum(ent).reshape(1, 1, 1)


def kernel(hidden_states, W_router, b_router, W1, b1, W2, b2):
    b, s, hd = hidden_states.shape
    n = b * s
    e = W_router.shape[0]
    h4 = W1.shape[0]
    x = hidden_states.reshape(n, hd)
    consts = _CONST_CACHE.get((n, e))
    if consts is None:
        consts = _routing_consts(n, e)
    g1, u2, g3, ex = consts

    t = _TOKEN_BLOCK
    grid = (n // t,)
    full = lambda i: (0, 0)
    tok = lambda i: (i, 0)
    out = pl.pallas_call(
        _routing_body,
        grid=grid,
        in_specs=[
            pl.BlockSpec((t, hd), tok),
            pl.BlockSpec((e, hd), full),
            pl.BlockSpec((1, e), full),
            pl.BlockSpec((h4, hd), full),
            pl.BlockSpec((1, h4), full),
            pl.BlockSpec((128, h4), full),
            pl.BlockSpec((1, 1), full),
            pl.BlockSpec((t, e), tok),
            pl.BlockSpec((t, e), tok),
            pl.BlockSpec((t, 1), tok),
            pl.BlockSpec((t, 1), tok),
        ],
        out_specs=[
            pl.BlockSpec((t, e), tok),
            pl.BlockSpec((t, 1), tok),
            pl.BlockSpec((t, e), tok),
            pl.BlockSpec((t, 1), tok),
            pl.BlockSpec((1, 1, 1), lambda i: (i, 0, 0)),
        ],
        out_shape=[
            jax.ShapeDtypeStruct((n, e), jnp.float32),
            jax.ShapeDtypeStruct((n, 1), jnp.int32),
            jax.ShapeDtypeStruct((n, e), jnp.float32),
            jax.ShapeDtypeStruct((n, 1), jnp.float32),
            jax.ShapeDtypeStruct((n // t, 1, 1), jnp.float32),
        ],
        compiler_params=pltpu.CompilerParams(
            dimension_semantics=("parallel",),
        ),
    )
    w2_pad = jnp.zeros((128, h4), jnp.float32).at[0].set(W2[0])
    out = out(x, W_router, b_router.reshape(1, e),
              W1, b1.reshape(1, h4),
              w2_pad, b2.reshape(1, 1), g1, g3, u2, ex)

    p, idx, noisy, explor, ent = out
    entropy = jnp.sum(ent) / n
    return (p, idx, noisy, explor[:, 0], entropy)


# speedup vs baseline: 1.0003x; 1.0003x over previous
import jax
import jax.numpy as jnp
import numpy as np
from jax.experimental import pallas as pl
from jax.experimental.pallas import tpu as pltpu

_TOKEN_BLOCK = 512


def _routing_consts(n: int, e: int):
    rk = jax.random.key(42)
    k1, k2, k3, k4 = jax.random.split(rk, 4)
    u = jax.random.uniform(k1, (n, e), dtype=jnp.float32)
    g1 = -jnp.log(-jnp.log(u + 1e-08) + 1e-08)
    u2 = jax.random.uniform(k2, (n,), jnp.float32)
    g3 = jax.random.gumbel(k3, (n, e), jnp.float32)
    ex = jax.random.randint(k4, (n,), 0, e)
    return (g1, u2.reshape(n, 1), g3, ex.reshape(n, 1).astype(jnp.int32))


_CONST_CACHE = {}
try:
    _CONST_CACHE[(8192, 64)] = tuple(
        np.asarray(a) for a in _routing_consts(8192, 64))
except Exception:
    pass


def _routing_body(x_ref, wr_ref, br_ref, w1_ref, b1_ref, w2_ref, b2_ref,
                  g1_ref, g3_ref, u2_ref, ex_ref,
                  w_out, idx_out, noisy_out, explor_out, ent_out):
    e = wr_ref.shape[0]
    xb = x_ref[...]
    dims = (((1,), (1,)), ((), ()))
    logits = jax.lax.dot_general(xb, wr_ref[...], dims,
                                 preferred_element_type=jnp.float32) + br_ref[...]
    h = jax.lax.dot_general(xb, w1_ref[...], dims,
                            preferred_element_type=jnp.float32) + b1_ref[...]
    h = jnp.maximum(h, 0.0)
    epre = jax.lax.dot_general(h, w2_ref[...], dims,
                               preferred_element_type=jnp.float32)[:, 0:1] + b2_ref[...]
    explor = jax.nn.sigmoid(epre)

    noisy = logits + g1_ref[...]
    m = jnp.max(noisy, axis=-1, keepdims=True)
    ee = jnp.exp(noisy - m)
    p = ee / jnp.sum(ee, axis=-1, keepdims=True)
    logp = jnp.log(p + 1e-08)
    ent = -jnp.sum(p * logp, axis=-1)

    score = logp + g3_ref[...]
    smax = jnp.max(score, axis=-1, keepdims=True)
    lane = jax.lax.broadcasted_iota(jnp.int32, score.shape, 1)
    exploit = jnp.min(jnp.where(score >= smax, lane, e), axis=-1, keepdims=True)
    mask = u2_ref[...] < 1.0 - explor * 0.1
    idx = jnp.where(mask, exploit, ex_ref[...])

    w_out[...] = p
    idx_out[...] = idx
    noisy_out[...] = noisy
    explor_out[...] = explor
    ent_out[...] = jnp.sum(ent).reshape(1, 1, 1)


def kernel(hidden_states, W_router, b_router, W1, b1, W2, b2):
    b, s, hd = hidden_states.shape
    n = b * s
    e = W_router.shape[0]
    h4 = W1.shape[0]
    x = hidden_states.reshape(n, hd)
    consts = _CONST_CACHE.get((n, e))
    if consts is None:
        consts = _routing_consts(n, e)
    g1, u2, g3, ex = consts

    t = _TOKEN_BLOCK
    grid = (n // t,)
    full = lambda i: (0, 0)
    tok = lambda i: (i, 0)
    out = pl.pallas_call(
        _routing_body,
        grid=grid,
        in_specs=[
            pl.BlockSpec((t, hd), tok),
            pl.BlockSpec((e, hd), full),
            pl.BlockSpec((1, e), full),
            pl.BlockSpec((h4, hd), full),
            pl.BlockSpec((1, h4), full),
            pl.BlockSpec((128, h4), full),
            pl.BlockSpec((1, 1), full),
            pl.BlockSpec((t, e), tok),
            pl.BlockSpec((t, e), tok),
            pl.BlockSpec((t, 1), tok),
            pl.BlockSpec((t, 1), tok),
        ],
        out_specs=[
            pl.BlockSpec((t, e), tok),
            pl.BlockSpec((t, 1), tok),
            pl.BlockSpec((t, e), tok),
            pl.BlockSpec((t, 1), tok),
            pl.BlockSpec((1, 1, 1), lambda i: (i, 0, 0)),
        ],
        out_shape=[
            jax.ShapeDtypeStruct((n, e), jnp.float32),
            jax.ShapeDtypeStruct((n, 1), jnp.int32),
            jax.ShapeDtypeStruct((n, e), jnp.float32),
            jax.ShapeDtypeStruct((n, 1), jnp.float32),
            jax.ShapeDtypeStruct((n // t, 1, 1), jnp.float32),
        ],
        compiler_params=pltpu.CompilerParams(
            dimension_semantics=("arbitrary",),
        ),
    )
    w2_pad = jnp.zeros((128, h4), jnp.float32).at[0].set(W2[0])
    out = out(x, W_router, b_router.reshape(1, e),
              W1, b1.reshape(1, h4),
              w2_pad, b2.reshape(1, 1), g1, g3, u2, ex)

    p, idx, noisy, explor, ent = out
    entropy = jnp.sum(ent) / n
    return (p, idx, noisy, explor[:, 0], entropy)


# speedup vs baseline: 1.0258x; 1.0255x over previous
import jax
import jax.numpy as jnp
import numpy as np
from jax.experimental import pallas as pl
from jax.experimental.pallas import tpu as pltpu

_TOKEN_BLOCK = 512


def _routing_consts(n: int, e: int):
    rk = jax.random.key(42)
    k1, k2, k3, k4 = jax.random.split(rk, 4)
    u = jax.random.uniform(k1, (n, e), dtype=jnp.float32)
    g1 = -jnp.log(-jnp.log(u + 1e-08) + 1e-08)
    u2 = jax.random.uniform(k2, (n,), jnp.float32)
    g3 = jax.random.gumbel(k3, (n, e), jnp.float32)
    ex = jax.random.randint(k4, (n,), 0, e)
    return (g1, u2.reshape(n, 1), g3, ex.reshape(n, 1).astype(jnp.int32))


_CONST_CACHE = {}
try:
    _CONST_CACHE[(8192, 64)] = tuple(
        np.asarray(a) for a in _routing_consts(8192, 64))
except Exception:
    pass


def _routing_body(x_ref, wr_ref, br_ref, w1_ref, b1_ref, w2_ref, b2_ref,
                  g1_ref, g3_ref, u2_ref, ex_ref,
                  w_out, idx_out, noisy_out, explor_out, ent_out):
    e = wr_ref.shape[0]
    xb = x_ref[...]
    dims = (((1,), (1,)), ((), ()))
    logits = jax.lax.dot_general(xb, wr_ref[...], dims,
                                 preferred_element_type=jnp.float32) + br_ref[...]
    h = jax.lax.dot_general(xb, w1_ref[...], dims,
                            preferred_element_type=jnp.float32) + b1_ref[...]
    h = jnp.maximum(h, 0.0)
    epre = jax.lax.dot_general(h, w2_ref[...], dims,
                               preferred_element_type=jnp.float32)[:, 0:1] + b2_ref[...]
    explor = jax.nn.sigmoid(epre)

    noisy = logits + g1_ref[...]
    m = jnp.max(noisy, axis=-1, keepdims=True)
    ee = jnp.exp(noisy - m)
    p = ee / jnp.sum(ee, axis=-1, keepdims=True)
    logp = jnp.log(p + 1e-08)
    ent = -jnp.sum(p * logp, axis=-1)

    score = logp + g3_ref[...]
    smax = jnp.max(score, axis=-1, keepdims=True)
    lane = jax.lax.broadcasted_iota(jnp.int32, score.shape, 1)
    exploit = jnp.min(jnp.where(score >= smax, lane, e), axis=-1, keepdims=True)
    mask = u2_ref[...] < 1.0 - explor * 0.1
    idx = jnp.where(mask, exploit, ex_ref[...])

    w_out[...] = p
    idx_out[...] = idx
    noisy_out[...] = noisy
    explor_out[...] = explor
    i = pl.program_id(0)
    part = jnp.sum(ent).reshape(1, 1, 1) * (1.0 / (pl.num_programs(0) * p.shape[0]))

    @pl.when(i == 0)
    def _():
        ent_out[...] = part

    @pl.when(i > 0)
    def _():
        ent_out[...] = ent_out[...] + part


def kernel(hidden_states, W_router, b_router, W1, b1, W2, b2):
    b, s, hd = hidden_states.shape
    n = b * s
    e = W_router.shape[0]
    h4 = W1.shape[0]
    x = hidden_states.reshape(n, hd)
    consts = _CONST_CACHE.get((n, e))
    if consts is None:
        consts = _routing_consts(n, e)
    g1, u2, g3, ex = consts

    t = _TOKEN_BLOCK
    grid = (n // t,)
    full = lambda i: (0, 0)
    tok = lambda i: (i, 0)
    out = pl.pallas_call(
        _routing_body,
        grid=grid,
        in_specs=[
            pl.BlockSpec((t, hd), tok),
            pl.BlockSpec((e, hd), full),
            pl.BlockSpec((1, e), full),
            pl.BlockSpec((h4, hd), full),
            pl.BlockSpec((1, h4), full),
            pl.BlockSpec((128, h4), full),
            pl.BlockSpec((1, 1), full),
            pl.BlockSpec((t, e), tok),
            pl.BlockSpec((t, e), tok),
            pl.BlockSpec((t, 1), tok),
            pl.BlockSpec((t, 1), tok),
        ],
        out_specs=[
            pl.BlockSpec((t, e), tok),
            pl.BlockSpec((t, 1), tok),
            pl.BlockSpec((t, e), tok),
            pl.BlockSpec((t, 1), tok),
            pl.BlockSpec((1, 1, 1), lambda i: (0, 0, 0)),
        ],
        out_shape=[
            jax.ShapeDtypeStruct((n, e), jnp.float32),
            jax.ShapeDtypeStruct((n, 1), jnp.int32),
            jax.ShapeDtypeStruct((n, e), jnp.float32),
            jax.ShapeDtypeStruct((n, 1), jnp.float32),
            jax.ShapeDtypeStruct((1, 1, 1), jnp.float32),
        ],
        compiler_params=pltpu.CompilerParams(
            dimension_semantics=("arbitrary",),
        ),
    )
    w2_pad = jnp.zeros((128, h4), jnp.float32).at[0].set(W2[0])
    out = out(x, W_router, b_router.reshape(1, e),
              W1, b1.reshape(1, h4),
              w2_pad, b2.reshape(1, 1), g1, g3, u2, ex)

    p, idx, noisy, explor, ent = out
    return (p, idx, noisy, explor[:, 0], ent[0, 0, 0])


# speedup vs baseline: 1.0323x; 1.0064x over previous
import jax
import jax.numpy as jnp
import numpy as np
from jax.experimental import pallas as pl
from jax.experimental.pallas import tpu as pltpu

_TOKEN_BLOCK = 512


def _routing_consts(n: int, e: int):
    rk = jax.random.key(42)
    k1, k2, k3, k4 = jax.random.split(rk, 4)
    u = jax.random.uniform(k1, (n, e), dtype=jnp.float32)
    g1 = -jnp.log(-jnp.log(u + 1e-08) + 1e-08)
    u2 = jax.random.uniform(k2, (n,), jnp.float32)
    g3 = jax.random.gumbel(k3, (n, e), jnp.float32)
    ex = jax.random.randint(k4, (n,), 0, e)
    return (g1, u2.reshape(n, 1), g3, ex.reshape(n, 1).astype(jnp.int32))


_CONST_CACHE = {}
try:
    _CONST_CACHE[(8192, 64)] = tuple(
        np.asarray(a) for a in _routing_consts(8192, 64))
except Exception:
    pass


def _routing_body(x_ref, wr_ref, br_ref, w1_ref, b1_ref, w2_ref, b2_ref,
                  g1_ref, g3_ref, u2_ref, ex_ref,
                  w_out, idx_out, noisy_out, explor_out, ent_out):
    e = wr_ref.shape[0]
    xb = x_ref[...]
    dims = (((1,), (1,)), ((), ()))
    logits = jax.lax.dot_general(xb, wr_ref[...], dims,
                                 preferred_element_type=jnp.float32) + br_ref[...]
    h = jax.lax.dot_general(xb, w1_ref[...], dims,
                            preferred_element_type=jnp.float32) + b1_ref[...]
    h = jnp.maximum(h, 0.0)
    epre = jax.lax.dot_general(h, w2_ref[...], dims,
                               preferred_element_type=jnp.float32)[:, 0:1] + b2_ref[...]
    explor = jax.nn.sigmoid(epre)

    noisy = logits + g1_ref[...]
    m = jnp.max(noisy, axis=-1, keepdims=True)
    ee = jnp.exp(noisy - m)
    p = ee / jnp.sum(ee, axis=-1, keepdims=True)
    logp = jnp.log(p + 1e-08)
    ent = -jnp.sum(p * logp, axis=-1)

    score = logp + g3_ref[...]
    smax = jnp.max(score, axis=-1, keepdims=True)
    lane = jax.lax.broadcasted_iota(jnp.int32, score.shape, 1)
    exploit = jnp.min(jnp.where(score >= smax, lane, e), axis=-1, keepdims=True)
    mask = u2_ref[...] < 1.0 - explor * 0.1
    idx = jnp.where(mask, exploit, ex_ref[...])

    w_out[...] = p
    idx_out[...] = idx
    noisy_out[...] = noisy
    explor_out[...] = explor[:, 0]
    i = pl.program_id(0)
    part = jnp.sum(ent).reshape(1, 1, 1) * (1.0 / (pl.num_programs(0) * p.shape[0]))

    @pl.when(i == 0)
    def _():
        ent_out[...] = part

    @pl.when(i > 0)
    def _():
        ent_out[...] = ent_out[...] + part


def kernel(hidden_states, W_router, b_router, W1, b1, W2, b2):
    b, s, hd = hidden_states.shape
    n = b * s
    e = W_router.shape[0]
    h4 = W1.shape[0]
    x = hidden_states.reshape(n, hd)
    consts = _CONST_CACHE.get((n, e))
    if consts is None:
        consts = _routing_consts(n, e)
    g1, u2, g3, ex = consts

    t = _TOKEN_BLOCK
    grid = (n // t,)
    full = lambda i: (0, 0)
    tok = lambda i: (i, 0)
    out = pl.pallas_call(
        _routing_body,
        grid=grid,
        in_specs=[
            pl.BlockSpec((t, hd), tok),
            pl.BlockSpec((e, hd), full),
            pl.BlockSpec((1, e), full),
            pl.BlockSpec((h4, hd), full),
            pl.BlockSpec((1, h4), full),
            pl.BlockSpec((128, h4), full),
            pl.BlockSpec((1, 1), full),
            pl.BlockSpec((t, e), tok),
            pl.BlockSpec((t, e), tok),
            pl.BlockSpec((t, 1), tok),
            pl.BlockSpec((t, 1), tok),
        ],
        out_specs=[
            pl.BlockSpec((t, e), tok),
            pl.BlockSpec((t, 1), tok),
            pl.BlockSpec((t, e), tok),
            pl.BlockSpec((t,), lambda i: (i,)),
            pl.BlockSpec((1, 1, 1), lambda i: (0, 0, 0)),
        ],
        out_shape=[
            jax.ShapeDtypeStruct((n, e), jnp.float32),
            jax.ShapeDtypeStruct((n, 1), jnp.int32),
            jax.ShapeDtypeStruct((n, e), jnp.float32),
            jax.ShapeDtypeStruct((n,), jnp.float32),
            jax.ShapeDtypeStruct((1, 1, 1), jnp.float32),
        ],
        compiler_params=pltpu.CompilerParams(
            dimension_semantics=("arbitrary",),
        ),
    )
    w2_pad = jnp.zeros((128, h4), jnp.float32).at[0].set(W2[0])
    out = out(x, W_router, b_router.reshape(1, e),
              W1, b1.reshape(1, h4),
              w2_pad, b2.reshape(1, 1), g1, g3, u2, ex)

    p, idx, noisy, explor, ent = out
    return (p, idx, noisy, explor, ent[0, 0, 0])
